# Initial kernel scaffold; baseline (speedup 1.0000x reference)
import jax
import jax.numpy as jnp
from jax.experimental import pallas as pl


def kernel(full_edge_0, full_edge_0_hat_logits, t_edge):
    raise NotImplementedError("write your pallas kernel here")


# speedup vs baseline: 23.6114x; 23.6114x over previous
import functools

import jax
import jax.numpy as jnp
from jax import lax
from jax.experimental import pallas as pl
from jax.experimental.pallas import tpu as pltpu
from jax.experimental.pallas import tpu_sc as plsc

E = 32768
NUM_T = 64
LBD = 0.1
L = 16

LN2 = 0.6931471805599453
SQRT2 = 1.4142135623730951


def _flog(x):
    xi = lax.bitcast_convert_type(x, jnp.int32)
    ex = jnp.bitwise_and(lax.shift_right_logical(xi, 23), 0xFF) - 127
    mb = jnp.bitwise_or(jnp.bitwise_and(xi, 0x007FFFFF), 0x3F800000)
    m = lax.bitcast_convert_type(mb, jnp.float32)
    big = m > SQRT2
    m = jnp.where(big, m * 0.5, m)
    ex = jnp.where(big, ex + 1, ex).astype(jnp.float32)
    s = (m - 1.0) / (m + 1.0)
    z = s * s
    poly = 1.0 + z * (1.0 / 3.0 + z * (1.0 / 5.0 + z * (1.0 / 7.0 + z * (1.0 / 9.0))))
    return 2.0 * s * poly + ex * LN2


def _flog1p(u):
    s = u / (2.0 + u)
    z = s * s
    poly = 1.0 + z * (1.0 / 3.0 + z * (1.0 / 5.0 + z * (1.0 / 7.0 + z * (1.0 / 9.0))))
    return 2.0 * s * poly


def _make_sc_kernel():
    info = plsc.get_sparse_core_info()
    nc, ns = info.num_cores, info.num_subcores
    nw = nc * ns
    chunk = E // nw
    nvec = chunk // L
    mesh = plsc.VectorSubcoreMesh(core_axis_name="c", subcore_axis_name="s")

    @functools.partial(
        pl.kernel,
        mesh=mesh,
        out_type=jax.ShapeDtypeStruct((nw, 3, L), jnp.float32),
        scratch_types=[
            pltpu.VMEM((chunk,), jnp.float32),
            pltpu.VMEM((chunk,), jnp.float32),
            pltpu.VMEM((chunk,), jnp.float32),
            pltpu.VMEM((chunk,), jnp.int32),
            pltpu.VMEM((3, L), jnp.float32),
        ],
    )
    def sc_kernel(x0_hbm, l0_hbm, l1_hbm, t_hbm, out_hbm, x0_v, l0_v, l1_v, t_v, part_v):
        wid = lax.axis_index("s") * nc + lax.axis_index("c")
        base = wid * chunk
        pltpu.sync_copy(x0_hbm.at[pl.ds(base, chunk)], x0_v)
        pltpu.sync_copy(l0_hbm.at[pl.ds(base, chunk)], l0_v)
        pltpu.sync_copy(l1_hbm.at[pl.ds(base, chunk)], l1_v)
        pltpu.sync_copy(t_hbm.at[pl.ds(base, chunk)], t_v)

        def step(i, carry):
            kl_a, ax_a, ac_a = carry
            sl = pl.ds(i * L, L)
            x0 = x0_v[sl]
            l0 = l0_v[sl]
            l1 = l1_v[sl]
            tf = t_v[sl].astype(jnp.float32)

            kt = 1.0 - tf * (1.0 / 64.0)
            ktm1 = kt + (1.0 / 64.0)
            bt = kt / ktm1

            s1 = 1.0 / (1.0 + jnp.exp(l0 - l1))
            s0 = 1.0 - s1

            xk = x0 * kt
            q0 = (1.0 - xk) * bt + (1.0 - bt)
            q1 = xk * bt
            pr0 = (s0 * ktm1 + (1.0 - ktm1)) * q0
            pr1 = (s1 * ktm1) * q1
            rs = 1.0 / (pr0 + pr1 + 1e-6)
            ftr = tf == 1.0
            tm10 = jnp.where(ftr, s0, pr0 * rs)
            tm11 = jnp.where(ftr, s1, pr1 * rs)
            u0 = ((1.0 - x0) * ktm1 + (1.0 - ktm1)) * q0
            u1 = (x0 * ktm1) * q1
            us = 1.0 / (u0 + u1 + 1e-6)
            g0 = jnp.clip(u0 * us, 0.0, 1.0)
            g1 = jnp.clip(u1 * us, 0.0, 1.0)
            p0 = jnp.clip(tm10, 1e-6, 1.0 - 1e-6)
            p1 = jnp.clip(tm11, 1e-6, 1.0 - 1e-6)
            aux = -(g0 * _flog(p0) + (1.0 - g0) * _flog(1.0 - p0)
                    + g1 * _flog(p1) + (1.0 - g1) * _flog(1.0 - p1))

            kl0 = jnp.maximum(l0, 0.0) - l0 * (1.0 - x0) + _flog1p(jnp.exp(-jnp.abs(l0)))
            kl1 = jnp.maximum(l1, 0.0) - l1 * x0 + _flog1p(jnp.exp(-jnp.abs(l1)))

            af = jnp.where(l1 > l0, 1.0, 0.0)
            accv = jnp.where(af == x0, 1.0, 0.0)
            return kl_a + (kl0 + kl1), ax_a + aux, ac_a + accv

        zero = jnp.zeros((L,), jnp.float32)
        kl_s, ax_s, ac_s = lax.fori_loop(0, nvec, step, (zero, zero, zero))
        part_v[0, :] = kl_s
        part_v[1, :] = ax_s
        part_v[2, :] = ac_s
        pltpu.sync_copy(part_v, out_hbm.at[wid])

    return sc_kernel


def kernel(full_edge_0, full_edge_0_hat_logits, t_edge):
    sc = _make_sc_kernel()
    l0 = full_edge_0_hat_logits[:, 0]
    l1 = full_edge_0_hat_logits[:, 1]
    parts = sc(full_edge_0, l0, l1, t_edge)
    sums = parts.sum(axis=(0, 2))
    kl_loss = sums[0] / (2.0 * E)
    aux_loss = sums[1] / (2.0 * E)
    acc = sums[2] / E
    total = LBD * aux_loss + kl_loss
    return (total, kl_loss, acc, aux_loss)


# speedup vs baseline: 24.7386x; 1.0477x over previous
import functools

import jax
import jax.numpy as jnp
from jax import lax
from jax.experimental import pallas as pl
from jax.experimental.pallas import tpu as pltpu
from jax.experimental.pallas import tpu_sc as plsc

E = 32768
NUM_T = 64
LBD = 0.1
L = 16

LN2 = 0.6931471805599453
SQRT2 = 1.4142135623730951


def _flog(x):
    xi = lax.bitcast_convert_type(x, jnp.int32)
    ex = jnp.bitwise_and(lax.shift_right_logical(xi, 23), 0xFF) - 127
    mb = jnp.bitwise_or(jnp.bitwise_and(xi, 0x007FFFFF), 0x3F800000)
    m = lax.bitcast_convert_type(mb, jnp.float32)
    big = m > SQRT2
    m = jnp.where(big, m * 0.5, m)
    ex = jnp.where(big, ex + 1, ex).astype(jnp.float32)
    s = (m - 1.0) / (m + 1.0)
    z = s * s
    poly = 1.0 + z * (1.0 / 3.0 + z * (1.0 / 5.0 + z * (1.0 / 7.0 + z * (1.0 / 9.0))))
    return 2.0 * s * poly + ex * LN2


def _flog1p(u):
    s = u / (2.0 + u)
    z = s * s
    poly = 1.0 + z * (1.0 / 3.0 + z * (1.0 / 5.0 + z * (1.0 / 7.0 + z * (1.0 / 9.0))))
    return 2.0 * s * poly


def _make_sc_kernel():
    info = plsc.get_sparse_core_info()
    nc, ns = info.num_cores, info.num_subcores
    nw = nc * ns
    chunk = E // nw
    nvec = chunk // L
    mesh = plsc.VectorSubcoreMesh(core_axis_name="c", subcore_axis_name="s")

    @functools.partial(
        pl.kernel,
        mesh=mesh,
        out_type=jax.ShapeDtypeStruct((nw, 3, L), jnp.float32),
        scratch_types=[
            pltpu.VMEM((chunk,), jnp.float32),
            pltpu.VMEM((chunk,), jnp.float32),
            pltpu.VMEM((chunk,), jnp.float32),
            pltpu.VMEM((chunk,), jnp.int32),
            pltpu.VMEM((3, L), jnp.float32),
            pltpu.SemaphoreType.DMA,
            pltpu.SemaphoreType.DMA,
            pltpu.SemaphoreType.DMA,
            pltpu.SemaphoreType.DMA,
        ],
    )
    def sc_kernel(x0_hbm, l0_hbm, l1_hbm, t_hbm, out_hbm, x0_v, l0_v, l1_v,
                  t_v, part_v, sem0, sem1, sem2, sem3):
        wid = lax.axis_index("s") * nc + lax.axis_index("c")
        base = wid * chunk
        cp0 = pltpu.async_copy(x0_hbm.at[pl.ds(base, chunk)], x0_v, sem0)
        cp1 = pltpu.async_copy(l0_hbm.at[pl.ds(base, chunk)], l0_v, sem1)
        cp2 = pltpu.async_copy(l1_hbm.at[pl.ds(base, chunk)], l1_v, sem2)
        cp3 = pltpu.async_copy(t_hbm.at[pl.ds(base, chunk)], t_v, sem3)
        cp0.wait()
        cp1.wait()
        cp2.wait()
        cp3.wait()

        def step(i, carry):
            kl_a, ax_a, ac_a = carry
            sl = pl.ds(i * L, L)
            x0 = x0_v[sl]
            l0 = l0_v[sl]
            l1 = l1_v[sl]
            tf = t_v[sl].astype(jnp.float32)

            kt = 1.0 - tf * (1.0 / 64.0)
            ktm1 = kt + (1.0 / 64.0)
            bt = kt / ktm1

            s1 = 1.0 / (1.0 + jnp.exp(l0 - l1))
            s0 = 1.0 - s1

            xk = x0 * kt
            q0 = (1.0 - xk) * bt + (1.0 - bt)
            q1 = xk * bt
            pr0 = (s0 * ktm1 + (1.0 - ktm1)) * q0
            pr1 = (s1 * ktm1) * q1
            rs = 1.0 / (pr0 + pr1 + 1e-6)
            ftr = tf == 1.0
            tm10 = jnp.where(ftr, s0, pr0 * rs)
            tm11 = jnp.where(ftr, s1, pr1 * rs)
            u0 = ((1.0 - x0) * ktm1 + (1.0 - ktm1)) * q0
            u1 = (x0 * ktm1) * q1
            us = 1.0 / (u0 + u1 + 1e-6)
            g0 = jnp.clip(u0 * us, 0.0, 1.0)
            g1 = jnp.clip(u1 * us, 0.0, 1.0)
            p0 = jnp.clip(tm10, 1e-6, 1.0 - 1e-6)
            p1 = jnp.clip(tm11, 1e-6, 1.0 - 1e-6)
            aux = -(g0 * _flog(p0) + (1.0 - g0) * _flog(1.0 - p0)
                    + g1 * _flog(p1) + (1.0 - g1) * _flog(1.0 - p1))

            kl0 = jnp.maximum(l0, 0.0) - l0 * (1.0 - x0) + _flog1p(jnp.exp(-jnp.abs(l0)))
            kl1 = jnp.maximum(l1, 0.0) - l1 * x0 + _flog1p(jnp.exp(-jnp.abs(l1)))

            af = jnp.where(l1 > l0, 1.0, 0.0)
            accv = jnp.where(af == x0, 1.0, 0.0)
            return kl_a + (kl0 + kl1), ax_a + aux, ac_a + accv

        zero = jnp.zeros((L,), jnp.float32)
        kl_s, ax_s, ac_s = plsc.parallel_loop(
            0, nvec, unroll=4, carry=(zero, zero, zero))(step)
        part_v[0, :] = kl_s
        part_v[1, :] = ax_s
        part_v[2, :] = ac_s
        pltpu.sync_copy(part_v, out_hbm.at[wid])

    return sc_kernel


def kernel(full_edge_0, full_edge_0_hat_logits, t_edge):
    sc = _make_sc_kernel()
    l0 = full_edge_0_hat_logits[:, 0]
    l1 = full_edge_0_hat_logits[:, 1]
    parts = sc(full_edge_0, l0, l1, t_edge)
    sums = parts.sum(axis=(0, 2))
    kl_loss = sums[0] / (2.0 * E)
    aux_loss = sums[1] / (2.0 * E)
    acc = sums[2] / E
    total = LBD * aux_loss + kl_loss
    return (total, kl_loss, acc, aux_loss)
